# baseline (device time: 54724 ns/iter reference)
import jax
import jax.numpy as jnp
from jax import lax
from jax.experimental import pallas as pl
from jax.experimental.pallas import tpu as pltpu

N_DEV = 4
B_PER = 2
SQ = 256
SKV = 256
H_PER = 4
DH = 64
D_MODEL = 512
HD = H_PER * DH
BLK = 64
SCALE = 0.125
BF16 = jnp.bfloat16


def kernel(x, Wq, K_ext, V_ext, Wo):
    wq_b = Wq.astype(BF16)
    wo_b = Wo.astype(BF16)

    def body(x_ref, wq_ref, k_hbm, v_hbm, wo_ref, out_ref,
             wq_all, wo_all, x_bf, k_tiles, v_tiles,
             copy_sems, kv_sems, send_sems, recv_sems):
        my_pos = lax.axis_index("i")

        kv_copies = {}
        for off in range(N_DEV):
            j = (my_pos + off) % N_DEV
            cps = []
            for hh in range(H_PER):
                h = j * H_PER + hh
                for b in range(B_PER):
                    bg = my_pos * B_PER + b
                    for t, src, dst in (
                        (0, k_hbm, k_tiles), (1, v_hbm, v_tiles)
                    ):
                        c = pltpu.make_async_copy(
                            src.at[bg, :, h, :],
                            dst.at[off, hh, b],
                            kv_sems.at[t, off, hh, b],
                        )
                        c.start()
                        cps.append(c)
            kv_copies[off] = cps

        barrier = pltpu.get_barrier_semaphore()
        for off in range(1, N_DEV):
            pl.semaphore_signal(
                barrier, inc=1,
                device_id=((my_pos + off) % N_DEV,),
                device_id_type=pl.DeviceIdType.MESH,
            )
        pl.semaphore_wait(barrier, N_DEV - 1)

        cq = pltpu.make_async_copy(wq_ref, wq_all.at[my_pos], copy_sems.at[0])
        co = pltpu.make_async_copy(wo_ref, wo_all.at[my_pos], copy_sems.at[1])
        cq.start()
        co.start()

        cq.wait()
        co.wait()
        rdmas = []
        for off in (3, 2, 1):
            dst = (my_pos + off) % N_DEV
            for t, buf in ((0, wq_all), (1, wo_all)):
                r = pltpu.make_async_remote_copy(
                    src_ref=buf.at[my_pos],
                    dst_ref=buf.at[my_pos],
                    send_sem=send_sems.at[t, off],
                    recv_sem=recv_sems.at[t, off],
                    device_id=(dst,),
                    device_id_type=pl.DeviceIdType.MESH,
                )
                r.start()
                rdmas.append(r)

        for b in range(B_PER):
            x_bf[b] = x_ref[b].astype(BF16)

        qb = lax.broadcasted_iota(jnp.int32, (SQ, SKV), 0) // BLK
        kb = lax.broadcasted_iota(jnp.int32, (SQ, SKV), 1) // BLK
        mask = kb <= qb

        for off in range(N_DEV):
            j = (my_pos + off) % N_DEV
            if off:
                sem_off = N_DEV - off
                for t, buf in ((0, wq_all), (1, wo_all)):
                    pltpu.make_async_remote_copy(
                        src_ref=buf.at[j],
                        dst_ref=buf.at[j],
                        send_sem=send_sems.at[t, sem_off],
                        recv_sem=recv_sems.at[t, sem_off],
                        device_id=(j,),
                        device_id_type=pl.DeviceIdType.MESH,
                    ).wait_recv()
            for c in kv_copies[off]:
                c.wait()
            wqj = wq_all[j]
            woj = wo_all[j]
            for b in range(B_PER):
                qj = jnp.dot(
                    x_bf[b], wqj, preferred_element_type=jnp.float32
                ).astype(BF16)
                accs = []
                for hh in range(H_PER):
                    q = qj[:, hh * DH:(hh + 1) * DH]
                    k = k_tiles[off, hh, b].astype(BF16)
                    s = lax.dot_general(
                        q, k, (((1,), (1,)), ((), ())),
                        preferred_element_type=jnp.float32,
                    ) * SCALE
                    s = jnp.where(mask, s, -1e9)
                    e = jnp.exp(s - jnp.max(s, axis=-1, keepdims=True))
                    w = (e / jnp.sum(e, axis=-1, keepdims=True)).astype(BF16)
                    v = v_tiles[off, hh, b].astype(BF16)
                    ctx = jnp.dot(
                        w, v, preferred_element_type=jnp.float32
                    ).astype(BF16)
                    accs.append(jnp.dot(
                        ctx, woj[hh * DH:(hh + 1) * DH, :],
                        preferred_element_type=jnp.float32,
                    ))
                contrib = (accs[0] + accs[1]) + (accs[2] + accs[3])
                if off == 0:
                    out_ref[b] = contrib
                else:
                    out_ref[b] = out_ref[b] + contrib

        for r in rdmas:
            r.wait_send()

    return pl.pallas_call(
        body,
        out_shape=jax.ShapeDtypeStruct((B_PER, SQ, D_MODEL), jnp.float32),
        in_specs=[
            pl.BlockSpec(memory_space=pltpu.VMEM),
            pl.BlockSpec(memory_space=pltpu.VMEM),
            pl.BlockSpec(memory_space=pltpu.MemorySpace.HBM),
            pl.BlockSpec(memory_space=pltpu.MemorySpace.HBM),
            pl.BlockSpec(memory_space=pltpu.VMEM),
        ],
        out_specs=pl.BlockSpec(memory_space=pltpu.VMEM),
        scratch_shapes=[
            pltpu.VMEM((N_DEV, D_MODEL, HD), BF16),
            pltpu.VMEM((N_DEV, HD, D_MODEL), BF16),
            pltpu.VMEM((B_PER, SQ, D_MODEL), BF16),
            pltpu.VMEM((N_DEV, H_PER, B_PER, SKV, DH), jnp.float32),
            pltpu.VMEM((N_DEV, H_PER, B_PER, SKV, DH), jnp.float32),
            pltpu.SemaphoreType.DMA((2,)),
            pltpu.SemaphoreType.DMA((2, N_DEV, H_PER, B_PER)),
            pltpu.SemaphoreType.DMA((2, N_DEV)),
            pltpu.SemaphoreType.DMA((2, N_DEV)),
        ],
        compiler_params=pltpu.CompilerParams(collective_id=0),
    )(x, wq_b, K_ext, V_ext, wo_b)


# device time: 31380 ns/iter; 1.7439x vs baseline; 1.7439x over previous
import jax
import jax.numpy as jnp
from jax import lax
from jax.experimental import pallas as pl
from jax.experimental.pallas import tpu as pltpu

N_DEV = 4
B_PER = 2
SQ = 256
SKV = 256
H_PER = 4
DH = 64
D_MODEL = 512
HD = H_PER * DH
BLK = 64
SCALE = 0.125
BF16 = jnp.bfloat16


def kernel(x, Wq, K_ext, V_ext, Wo):
    my = lax.axis_index("i")
    k_loc = jnp.transpose(
        lax.dynamic_slice_in_dim(K_ext, my * B_PER, B_PER, axis=0).astype(BF16),
        (2, 0, 1, 3),
    )
    v_loc = jnp.transpose(
        lax.dynamic_slice_in_dim(V_ext, my * B_PER, B_PER, axis=0).astype(BF16),
        (2, 0, 1, 3),
    )
    wq_b = Wq.astype(BF16)
    wo_b = Wo.astype(BF16)

    def body(x_ref, wq_ref, k_ref, v_ref, wo_ref, out_ref,
             wq_all, wo_all, x_bf, copy_sems, send_sems, recv_sems):
        my_pos = lax.axis_index("i")

        barrier = pltpu.get_barrier_semaphore()
        for off in range(1, N_DEV):
            pl.semaphore_signal(
                barrier, inc=1,
                device_id=((my_pos + off) % N_DEV,),
                device_id_type=pl.DeviceIdType.MESH,
            )
        pl.semaphore_wait(barrier, N_DEV - 1)

        cq = pltpu.make_async_copy(wq_ref, wq_all.at[my_pos], copy_sems.at[0])
        co = pltpu.make_async_copy(wo_ref, wo_all.at[my_pos], copy_sems.at[1])
        cq.start()
        co.start()
        cq.wait()
        co.wait()

        rdmas = []
        for off in (3, 2, 1):
            dst = (my_pos + off) % N_DEV
            for t, buf in ((0, wq_all), (1, wo_all)):
                r = pltpu.make_async_remote_copy(
                    src_ref=buf.at[my_pos],
                    dst_ref=buf.at[my_pos],
                    send_sem=send_sems.at[t, off],
                    recv_sem=recv_sems.at[t, off],
                    device_id=(dst,),
                    device_id_type=pl.DeviceIdType.MESH,
                )
                r.start()
                rdmas.append(r)

        for b in range(B_PER):
            x_bf[b] = x_ref[b].astype(BF16)

        qb = lax.broadcasted_iota(jnp.int32, (SQ, SKV), 0) // BLK
        kb = lax.broadcasted_iota(jnp.int32, (SQ, SKV), 1) // BLK
        mask = kb <= qb

        for off in range(N_DEV):
            j = (my_pos + off) % N_DEV
            if off:
                sem_off = N_DEV - off
                for t, buf in ((0, wq_all), (1, wo_all)):
                    pltpu.make_async_remote_copy(
                        src_ref=buf.at[j],
                        dst_ref=buf.at[j],
                        send_sem=send_sems.at[t, sem_off],
                        recv_sem=recv_sems.at[t, sem_off],
                        device_id=(j,),
                        device_id_type=pl.DeviceIdType.MESH,
                    ).wait_recv()
            wqj = wq_all[j]
            woj = wo_all[j]
            for b in range(B_PER):
                qj = jnp.dot(
                    x_bf[b], wqj, preferred_element_type=jnp.float32
                ).astype(BF16)
                accs = []
                for hh in range(H_PER):
                    h = j * H_PER + hh
                    q = qj[:, hh * DH:(hh + 1) * DH]
                    s = lax.dot_general(
                        q, k_ref[h, b], (((1,), (1,)), ((), ())),
                        preferred_element_type=jnp.float32,
                    ) * SCALE
                    s = jnp.where(mask, s, -1e9)
                    e = jnp.exp(s - jnp.max(s, axis=-1, keepdims=True))
                    w = (e / jnp.sum(e, axis=-1, keepdims=True)).astype(BF16)
                    ctx = jnp.dot(
                        w, v_ref[h, b], preferred_element_type=jnp.float32
                    ).astype(BF16)
                    accs.append(jnp.dot(
                        ctx, woj[hh * DH:(hh + 1) * DH, :],
                        preferred_element_type=jnp.float32,
                    ))
                contrib = (accs[0] + accs[1]) + (accs[2] + accs[3])
                if off == 0:
                    out_ref[b] = contrib
                else:
                    out_ref[b] = out_ref[b] + contrib

        for r in rdmas:
            r.wait_send()

    return pl.pallas_call(
        body,
        out_shape=jax.ShapeDtypeStruct((B_PER, SQ, D_MODEL), jnp.float32),
        in_specs=[pl.BlockSpec(memory_space=pltpu.VMEM)] * 5,
        out_specs=pl.BlockSpec(memory_space=pltpu.VMEM),
        scratch_shapes=[
            pltpu.VMEM((N_DEV, D_MODEL, HD), BF16),
            pltpu.VMEM((N_DEV, HD, D_MODEL), BF16),
            pltpu.VMEM((B_PER, SQ, D_MODEL), BF16),
            pltpu.SemaphoreType.DMA((2,)),
            pltpu.SemaphoreType.DMA((2, N_DEV)),
            pltpu.SemaphoreType.DMA((2, N_DEV)),
        ],
        compiler_params=pltpu.CompilerParams(collective_id=0),
    )(x, wq_b, k_loc, v_loc, wo_b)
